# baseline (speedup 1.0000x reference)
import functools

import jax
import jax.numpy as jnp
from jax import lax
from jax.experimental import pallas as pl
from jax.experimental.pallas import tpu as pltpu
from jax.experimental.pallas import tpu_sc as plsc

B = 16384
D = 16
NC = 2
NS = 16
NW = NC * NS
CHUNK = 128
PER_W = B // NW
NCHUNK = PER_W // CHUNK
ROWS_PER_W = NCHUNK


def _sc_body(u_hbm, i_hbm, f_hbm, ub_hbm, uv_hbm, ib_hbm, iv_hbm, par_hbm,
             out_hbm,
             idxu_v, idxi_v, f_v, bu_v, bi_v, vu_v, vi_v, out_v, par_v, sem):
    wid = lax.axis_index("s") * NC + lax.axis_index("c")
    base = wid * PER_W
    row0 = wid * ROWS_PER_W

    pltpu.sync_copy(u_hbm.at[pl.ds(row0, ROWS_PER_W)], idxu_v)
    pltpu.sync_copy(i_hbm.at[pl.ds(row0, ROWS_PER_W)], idxi_v)
    pltpu.sync_copy(f_hbm.at[pl.ds(base, PER_W)], f_v)
    pltpu.sync_copy(par_hbm, par_v)

    copies = []
    for j in range(NCHUNK):
        dst = pl.ds(j * CHUNK, CHUNK)
        copies.append(pltpu.async_copy(ub_hbm.at[idxu_v.at[j]], bu_v.at[dst], sem))
        copies.append(pltpu.async_copy(uv_hbm.at[idxu_v.at[j]], vu_v.at[dst], sem))
        copies.append(pltpu.async_copy(ib_hbm.at[idxi_v.at[j]], bi_v.at[dst], sem))
        copies.append(pltpu.async_copy(iv_hbm.at[idxi_v.at[j]], vi_v.at[dst], sem))
    for c in copies:
        c.wait()

    w0 = par_v[0]
    w1 = par_v[1]
    cb = par_v[2]
    lanes = jnp.arange(16, dtype=jnp.int32)

    def group(g, _):
        off = g * 16
        r = lanes + off
        acc = jnp.zeros((16,), jnp.float32)
        for d in range(D):
            cd = jnp.full((16,), d, dtype=jnp.int32)
            acc = acc + plsc.load_gather(vu_v, [r, cd]) * plsc.load_gather(vi_v, [r, cd])
        fg = f_v[pl.ds(off, 16)]
        out_v[pl.ds(off, 16)] = (cb + bu_v[pl.ds(off, 16)] + bi_v[pl.ds(off, 16)]
                                 + acc + (w0 + w1 * fg) * fg)
        return 0

    lax.fori_loop(0, PER_W // 16, group, 0)
    pltpu.sync_copy(out_v, out_hbm.at[pl.ds(base, PER_W)])


@jax.jit
def _mfpoly2_sc(u2, i2, f, ub, uv, ib, iv, params):
    mesh = plsc.VectorSubcoreMesh(core_axis_name="c", subcore_axis_name="s")
    k = functools.partial(
        pl.kernel,
        out_type=jax.ShapeDtypeStruct((B,), jnp.float32),
        mesh=mesh,
        compiler_params=pltpu.CompilerParams(
            needs_layout_passes=False, use_tc_tiling_on_sc=False),
        scratch_types=[
            pltpu.VMEM((ROWS_PER_W, CHUNK), jnp.int32),
            pltpu.VMEM((ROWS_PER_W, CHUNK), jnp.int32),
            pltpu.VMEM((PER_W,), jnp.float32),
            pltpu.VMEM((PER_W,), jnp.float32),
            pltpu.VMEM((PER_W,), jnp.float32),
            pltpu.VMEM((PER_W, D), jnp.float32),
            pltpu.VMEM((PER_W, D), jnp.float32),
            pltpu.VMEM((PER_W,), jnp.float32),
            pltpu.VMEM((3, 16), jnp.float32),
            pltpu.SemaphoreType.DMA,
        ],
    )(_sc_body)
    return k(u2, i2, f, ub, uv, ib, iv, params)


def kernel(u, i, f, user_bias, user_vect, item_bias, item_vect, glob_bias,
           W_frame, b_frame):
    u2 = u.reshape(B // CHUNK, CHUNK).astype(jnp.int32)
    i2 = i.reshape(B // CHUNK, CHUNK).astype(jnp.int32)
    ub = user_bias.reshape(-1)
    ib = item_bias.reshape(-1)
    w = W_frame.reshape(2)
    cb = glob_bias[0] + b_frame[0]
    params = jnp.stack([
        jnp.full((16,), w[0], jnp.float32),
        jnp.full((16,), w[1], jnp.float32),
        jnp.full((16,), cb, jnp.float32),
    ])
    return _mfpoly2_sc(u2, i2, f, ub, user_vect, ib, item_vect, params)
